# Initial kernel scaffold; baseline (speedup 1.0000x reference)
import jax
import jax.numpy as jnp
from jax.experimental import pallas as pl


def kernel(user_emb, video_emb, publisher_emb, tag_emb, edge_index_uv, edge_index_up, edge_index_vu, edge_index_vt, edge_index_pu, edge_index_pt, edge_index_tv, edge_index_tp):
    raise NotImplementedError("write your pallas kernel here")


# speedup vs baseline: 2.6821x; 2.6821x over previous
import functools

import jax
import jax.numpy as jnp
from jax import lax
from jax.experimental import pallas as pl
from jax.experimental.pallas import tpu as pltpu
from jax.experimental.pallas import tpu_sc as plsc

N = 50000
EMB = 128
H = EMB // 2
HC = H // 2
E = 500000

NTILE = 16
NP = 50048
RPT = NP // NTILE
RB = 136
NBLK = RPT // RB
C = 192
NCHUNK = 163
EPT = NCHUNK * C
EP = EPT * NTILE
PADE = EP - E

ETYPES = ('uv', 'up', 'vu', 'vt', 'pu', 'pt', 'tv', 'tp')
REV = {'uv': 'vu', 'up': 'pu', 'vu': 'uv', 'vt': 'tv',
       'pu': 'up', 'pt': 'tp', 'tv': 'vt', 'tp': 'pt'}
COL_OFF = {'uv': 0, 'up': H, 'vu': 0, 'vt': H,
           'pu': 0, 'pt': H, 'tv': 0, 'tp': H}

_mesh = plsc.VectorSubcoreMesh(core_axis_name="c", subcore_axis_name="s")

_out_types = ([jax.ShapeDtypeStruct((2, NP, HC), jnp.float32)] * 8 +
              [jax.ShapeDtypeStruct((2, NP, HC), jnp.float32)] * 8)

_scratch = (
    [pltpu.VMEM_SHARED((NP, HC), jnp.float32),
     pltpu.VMEM_SHARED((NP,), jnp.float32)] +
    [pltpu.VMEM((C,), jnp.int32),
     pltpu.VMEM((C,), jnp.int32),
     pltpu.VMEM((C, HC), jnp.float32),
     pltpu.VMEM((C,), jnp.float32),
     pltpu.VMEM((RB, HC), jnp.float32),
     pltpu.VMEM((RB, HC), jnp.float32),
     pltpu.VMEM((RB, HC), jnp.float32),
     pltpu.VMEM((RB + 16,), jnp.float32),
     pltpu.VMEM((RB,), jnp.float32),
     pltpu.SemaphoreType.DMA]
)


@functools.partial(pl.kernel, mesh=_mesh, out_type=_out_types,
                   scratch_types=_scratch,
                   compiler_params=pltpu.CompilerParams(
                       use_tc_tiling_on_sc=False))
def _sc_aggr(*refs):
    tabs = refs[0:8]
    srcs = refs[8:16]
    dsts = refs[16:24]
    ones_h, zrows_h, zrpt_h = refs[24:27]
    out1 = refs[27:35]
    fin = refs[35:43]
    acc, cnt = refs[43:45]
    (sidx, didx, rows, ones_v, accb, bb, zb, rcb, z1b, sem) = refs[45:]

    c = lax.axis_index("c")
    s = lax.axis_index("s")
    row0 = s * RPT
    ebase = s * EPT

    pltpu.sync_copy(zrows_h, zb)
    pltpu.sync_copy(zrpt_h, z1b)
    pltpu.sync_copy(ones_h, ones_v)

    def zero_blk(b, _):
        r0 = row0 + b * RB
        pltpu.sync_copy(zb, acc.at[pl.ds(r0, RB), :])
        pltpu.sync_copy(z1b, cnt.at[pl.ds(r0, RB)])
        return 0
    lax.fori_loop(0, NBLK, zero_blk, 0)
    plsc.subcore_barrier()

    for layer in (1, 2):
        for ei, e in enumerate(ETYPES):
            rev_i = ETYPES.index(REV[e])
            tab = tabs[ei] if layer == 1 else out1[rev_i]

            def gs_body(i, _, srcr=srcs[ei], dstr=dsts[ei], tab=tab):
                off = ebase + i * C
                pltpu.sync_copy(srcr.at[pl.ds(off, C)], sidx)
                pltpu.sync_copy(dstr.at[pl.ds(off, C)], didx)
                pltpu.async_copy(tab.at[c].at[sidx], rows, sem).wait()
                pltpu.sync_copy(rows, acc.at[didx], add=True)
                pltpu.sync_copy(ones_v, cnt.at[didx], add=True)
                return 0
            lax.fori_loop(0, NCHUNK, gs_body, 0)
            plsc.subcore_barrier()

            def scale_blk(b, _, ei=ei, rev_i=rev_i, layer=layer):
                r0 = row0 + b * RB
                pltpu.sync_copy(cnt.at[pl.ds(r0, RB)], rcb.at[pl.ds(0, RB)])
                pltpu.sync_copy(acc.at[pl.ds(r0, RB), :], accb)
                if layer == 1:
                    pltpu.sync_copy(tabs[rev_i].at[c].at[pl.ds(r0, RB), :], bb)
                else:
                    pltpu.sync_copy(fin[ei].at[c].at[pl.ds(r0, RB), :], bb)

                def rcp_body(j, _):
                    v = rcb[pl.ds(j * 16, 16)]
                    rcb[pl.ds(j * 16, 16)] = 1.0 / jnp.maximum(v, 1.0)
                    return 0
                lax.fori_loop(0, (RB + 15) // 16, rcp_body, 0)

                def row_body(r, _, layer=layer):
                    rcv = rcb[pl.ds(r, 16)][0]
                    if layer == 1:
                        bc = jnp.broadcast_to(rcv, (16,))
                        for h in (0, 16):
                            m = accb[r, pl.ds(h, 16)] * bc
                            accb[r, pl.ds(h, 16)] = m
                            bb[r, pl.ds(h, 16)] = bb[r, pl.ds(h, 16)] + m * 0.5
                    else:
                        bc3 = jnp.broadcast_to(rcv * (1.0 / 3.0), (16,))
                        for h in (0, 16):
                            a = accb[r, pl.ds(h, 16)]
                            bb[r, pl.ds(h, 16)] = bb[r, pl.ds(h, 16)] + a * bc3
                    return 0
                lax.fori_loop(0, RB, row_body, 0)

                if layer == 1:
                    pltpu.sync_copy(accb, out1[ei].at[c].at[pl.ds(r0, RB), :])
                pltpu.sync_copy(bb, fin[ei].at[c].at[pl.ds(r0, RB), :])
                pltpu.sync_copy(zb, acc.at[pl.ds(r0, RB), :])
                pltpu.sync_copy(z1b, cnt.at[pl.ds(r0, RB)])
                return 0
            lax.fori_loop(0, NBLK, scale_blk, 0)
            plsc.subcore_barrier()


def kernel(user_emb, video_emb, publisher_emb, tag_emb,
           edge_index_uv, edge_index_up, edge_index_vu, edge_index_vt,
           edge_index_pu, edge_index_pt, edge_index_tv, edge_index_tp):
    embs = {'u': user_emb, 'v': video_emb, 'p': publisher_emb, 't': tag_emb}
    ei = {'uv': edge_index_uv, 'up': edge_index_up, 'vu': edge_index_vu,
          'vt': edge_index_vt, 'pu': edge_index_pu, 'pt': edge_index_pt,
          'tv': edge_index_tv, 'tp': edge_index_tp}

    pad_src = (jnp.arange(PADE, dtype=jnp.int32) % N)
    pad_dst = N + (jnp.arange(PADE, dtype=jnp.int32) % (NP - N))

    tabs, srcs, dsts = [], [], []
    for e in ETYPES:
        S = embs[e[0]]
        off = COL_OFF[e]
        t = jnp.stack([S[:, off:off + HC], S[:, off + HC:off + 2 * HC]])
        tabs.append(jnp.pad(t, ((0, 0), (0, NP - N), (0, 0))))
        srcs.append(jnp.concatenate([ei[e][0], pad_src]))
        dsts.append(jnp.concatenate([ei[e][1], pad_dst]))

    ones_c = jnp.ones((C,), jnp.float32)
    zrows = jnp.zeros((RB, HC), jnp.float32)
    zrpt = jnp.zeros((RB,), jnp.float32)

    outs = _sc_aggr(*tabs, *srcs, *dsts, ones_c, zrows, zrpt)
    fin = {e: outs[8 + i] for i, e in enumerate(ETYPES)}

    def cat(e1, e2):
        return jnp.concatenate([fin[e1][0, :N], fin[e1][1, :N],
                                fin[e2][0, :N], fin[e2][1, :N]], axis=1)

    return (cat('vu', 'pu'), cat('uv', 'tv'), cat('up', 'tp'),
            cat('vt', 'pt'))


# speedup vs baseline: 3.5128x; 1.3097x over previous
import functools

import jax
import jax.numpy as jnp
from jax import lax
from jax.experimental import pallas as pl
from jax.experimental.pallas import tpu as pltpu
from jax.experimental.pallas import tpu_sc as plsc

N = 50000
EMB = 128
H = EMB // 2
HC = H // 2
E = 500000

NTILE = 16
NP = 50048
RPT = NP // NTILE
RB = 136
NBLK = RPT // RB
C = 192
NCHUNK = 164
EPT = NCHUNK * C
EP = EPT * NTILE
PADE = EP - E

ETYPES = ('uv', 'up', 'vu', 'vt', 'pu', 'pt', 'tv', 'tp')
REV = {'uv': 'vu', 'up': 'pu', 'vu': 'uv', 'vt': 'tv',
       'pu': 'up', 'pt': 'tp', 'tv': 'vt', 'tp': 'pt'}
COL_OFF = {'uv': 0, 'up': H, 'vu': 0, 'vt': H,
           'pu': 0, 'pt': H, 'tv': 0, 'tp': H}

_mesh = plsc.VectorSubcoreMesh(core_axis_name="c", subcore_axis_name="s")

_out_types = ([jax.ShapeDtypeStruct((2, NP, HC), jnp.float32)] * 8 +
              [jax.ShapeDtypeStruct((2, NP, HC), jnp.float32)] * 8)

_scratch = (
    [pltpu.VMEM_SHARED((NP, HC), jnp.float32),
     pltpu.VMEM_SHARED((NP,), jnp.float32)] +
    [pltpu.VMEM((C,), jnp.int32),
     pltpu.VMEM((C,), jnp.int32),
     pltpu.VMEM((C, HC), jnp.float32),
     pltpu.VMEM((C,), jnp.int32),
     pltpu.VMEM((C,), jnp.int32),
     pltpu.VMEM((C, HC), jnp.float32),
     pltpu.VMEM((C,), jnp.float32),
     pltpu.VMEM((RB, HC), jnp.float32),
     pltpu.VMEM((RB, HC), jnp.float32),
     pltpu.VMEM((RB, HC), jnp.float32),
     pltpu.VMEM((RB + 16,), jnp.float32),
     pltpu.VMEM((RB,), jnp.float32),
     pltpu.SemaphoreType.DMA,
     pltpu.SemaphoreType.DMA]
)


@functools.partial(pl.kernel, mesh=_mesh, out_type=_out_types,
                   scratch_types=_scratch,
                   compiler_params=pltpu.CompilerParams(
                       use_tc_tiling_on_sc=False))
def _sc_aggr(*refs):
    tabs = refs[0:8]
    srcs = refs[8:16]
    dsts = refs[16:24]
    ones_h, zrows_h, zrpt_h = refs[24:27]
    out1 = refs[27:35]
    fin = refs[35:43]
    acc, cnt = refs[43:45]
    (sidx0, didx0, rows0, sidx1, didx1, rows1, ones_v,
     accb, bb, zb, rcb, z1b, sem0, sem1) = refs[45:]

    c = lax.axis_index("c")
    s = lax.axis_index("s")
    row0 = s * RPT
    ebase = s * EPT

    pltpu.sync_copy(zrows_h, zb)
    pltpu.sync_copy(zrpt_h, z1b)
    pltpu.sync_copy(ones_h, ones_v)

    def zero_blk(b, _):
        r0 = row0 + b * RB
        pltpu.sync_copy(zb, acc.at[pl.ds(r0, RB), :])
        pltpu.sync_copy(z1b, cnt.at[pl.ds(r0, RB)])
        return 0
    lax.fori_loop(0, NBLK, zero_blk, 0)
    plsc.subcore_barrier()

    for layer in (1, 2):
        for ei, e in enumerate(ETYPES):
            rev_i = ETYPES.index(REV[e])
            tab = tabs[ei] if layer == 1 else out1[rev_i]

            srcr, dstr = srcs[ei], dsts[ei]

            def load_idx(i, sidx, didx):
                off = ebase + i * C
                pltpu.sync_copy(srcr.at[pl.ds(off, C)], sidx)
                pltpu.sync_copy(dstr.at[pl.ds(off, C)], didx)

            def start_g(sidx, rows, sem, tab=tab):
                return pltpu.async_copy(tab.at[c].at[sidx], rows, sem)

            def drain(sidx, didx, rows, sem, tab=tab):
                pltpu.make_async_copy(tab.at[c].at[sidx], rows, sem).wait()
                pltpu.sync_copy(rows, acc.at[didx], add=True)
                pltpu.sync_copy(ones_v, cnt.at[didx], add=True)

            load_idx(0, sidx0, didx0)
            start_g(sidx0, rows0, sem0)

            def gs_pair(k, _):
                load_idx(2 * k + 1, sidx1, didx1)
                start_g(sidx1, rows1, sem1)
                drain(sidx0, didx0, rows0, sem0)

                @pl.when(2 * k + 2 < NCHUNK)
                def _():
                    load_idx(2 * k + 2, sidx0, didx0)
                    start_g(sidx0, rows0, sem0)
                drain(sidx1, didx1, rows1, sem1)
                return 0
            lax.fori_loop(0, NCHUNK // 2, gs_pair, 0)
            plsc.subcore_barrier()

            def scale_blk(b, _, ei=ei, rev_i=rev_i, layer=layer):
                r0 = row0 + b * RB
                pltpu.sync_copy(cnt.at[pl.ds(r0, RB)], rcb.at[pl.ds(0, RB)])
                pltpu.sync_copy(acc.at[pl.ds(r0, RB), :], accb)
                if layer == 1:
                    pltpu.sync_copy(tabs[rev_i].at[c].at[pl.ds(r0, RB), :], bb)
                else:
                    pltpu.sync_copy(fin[ei].at[c].at[pl.ds(r0, RB), :], bb)

                def rcp_body(j, _):
                    v = rcb[pl.ds(j * 16, 16)]
                    rcb[pl.ds(j * 16, 16)] = 1.0 / jnp.maximum(v, 1.0)
                    return 0
                lax.fori_loop(0, (RB + 15) // 16, rcp_body, 0)

                def row_body(r, _, layer=layer):
                    rcv = rcb[pl.ds(r, 16)][0]
                    if layer == 1:
                        bc = jnp.broadcast_to(rcv, (16,))
                        for h in (0, 16):
                            m = accb[r, pl.ds(h, 16)] * bc
                            accb[r, pl.ds(h, 16)] = m
                            bb[r, pl.ds(h, 16)] = bb[r, pl.ds(h, 16)] + m * 0.5
                    else:
                        bc3 = jnp.broadcast_to(rcv * (1.0 / 3.0), (16,))
                        for h in (0, 16):
                            a = accb[r, pl.ds(h, 16)]
                            bb[r, pl.ds(h, 16)] = bb[r, pl.ds(h, 16)] + a * bc3
                    return 0
                lax.fori_loop(0, RB, row_body, 0)

                if layer == 1:
                    pltpu.sync_copy(accb, out1[ei].at[c].at[pl.ds(r0, RB), :])
                pltpu.sync_copy(bb, fin[ei].at[c].at[pl.ds(r0, RB), :])
                pltpu.sync_copy(zb, acc.at[pl.ds(r0, RB), :])
                pltpu.sync_copy(z1b, cnt.at[pl.ds(r0, RB)])
                return 0
            lax.fori_loop(0, NBLK, scale_blk, 0)
            plsc.subcore_barrier()


def kernel(user_emb, video_emb, publisher_emb, tag_emb,
           edge_index_uv, edge_index_up, edge_index_vu, edge_index_vt,
           edge_index_pu, edge_index_pt, edge_index_tv, edge_index_tp):
    embs = {'u': user_emb, 'v': video_emb, 'p': publisher_emb, 't': tag_emb}
    ei = {'uv': edge_index_uv, 'up': edge_index_up, 'vu': edge_index_vu,
          'vt': edge_index_vt, 'pu': edge_index_pu, 'pt': edge_index_pt,
          'tv': edge_index_tv, 'tp': edge_index_tp}

    pad_src = (jnp.arange(PADE, dtype=jnp.int32) % N)
    pad_dst = N + (jnp.arange(PADE, dtype=jnp.int32) % (NP - N))

    tabs, srcs, dsts = [], [], []
    for e in ETYPES:
        S = embs[e[0]]
        off = COL_OFF[e]
        t = jnp.stack([S[:, off:off + HC], S[:, off + HC:off + 2 * HC]])
        tabs.append(jnp.pad(t, ((0, 0), (0, NP - N), (0, 0))))
        srcs.append(jnp.concatenate([ei[e][0], pad_src]))
        dsts.append(jnp.concatenate([ei[e][1], pad_dst]))

    ones_c = jnp.ones((C,), jnp.float32)
    zrows = jnp.zeros((RB, HC), jnp.float32)
    zrpt = jnp.zeros((RB,), jnp.float32)

    outs = _sc_aggr(*tabs, *srcs, *dsts, ones_c, zrows, zrpt)
    fin = {e: outs[8 + i] for i, e in enumerate(ETYPES)}

    def cat(e1, e2):
        return jnp.concatenate([fin[e1][0, :N], fin[e1][1, :N],
                                fin[e2][0, :N], fin[e2][1, :N]], axis=1)

    return (cat('vu', 'pu'), cat('uv', 'tv'), cat('up', 'tp'),
            cat('vt', 'pt'))


# speedup vs baseline: 5.9960x; 1.7069x over previous
import functools

import jax
import jax.numpy as jnp
from jax import lax
from jax.experimental import pallas as pl
from jax.experimental.pallas import tpu as pltpu
from jax.experimental.pallas import tpu_sc as plsc

N = 50000
EMB = 128
H = EMB // 2
HC = H // 2
E = 500000

NTILE = 16
NP = 50048
RPT = NP // NTILE
RB = 136
NBLK = RPT // RB
C = 304
NCHUNK = 104
EPT = NCHUNK * C
EP = EPT * NTILE
PADE = EP - E

ETYPES = ('uv', 'up', 'vu', 'vt', 'pu', 'pt', 'tv', 'tp')
REV = {'uv': 'vu', 'up': 'pu', 'vu': 'uv', 'vt': 'tv',
       'pu': 'up', 'pt': 'tp', 'tv': 'vt', 'tp': 'pt'}
COL_OFF = {'uv': 0, 'up': H, 'vu': 0, 'vt': H,
           'pu': 0, 'pt': H, 'tv': 0, 'tp': H}
NTYPES = ('u', 'v', 'p', 't')

_mesh = plsc.VectorSubcoreMesh(core_axis_name="c", subcore_axis_name="s")

_out_types = ([jax.ShapeDtypeStruct((2, NP, HC), jnp.float32)] * 8 +
              [jax.ShapeDtypeStruct((2, NP, HC), jnp.float32)] * 8)

_scratch = (
    [pltpu.VMEM_SHARED((NP, HC), jnp.float32),
     pltpu.VMEM_SHARED((NP,), jnp.float32)] +
    [pltpu.VMEM((2, C), jnp.int32),
     pltpu.VMEM((2, C), jnp.int32),
     pltpu.VMEM((C,), jnp.int32),
     pltpu.VMEM((C,), jnp.int32),
     pltpu.VMEM((C, HC), jnp.float32),
     pltpu.VMEM((C, HC), jnp.float32),
     pltpu.VMEM((C,), jnp.float32),
     pltpu.VMEM((RB, HC), jnp.float32),
     pltpu.VMEM((RB + 16,), jnp.float32),
     pltpu.VMEM((RB,), jnp.float32),
     pltpu.VMEM((RB,), jnp.int32),
     pltpu.SemaphoreType.DMA,
     pltpu.SemaphoreType.DMA]
)


@functools.partial(pl.kernel, mesh=_mesh, out_type=_out_types,
                   scratch_types=_scratch,
                   compiler_params=pltpu.CompilerParams(
                       use_tc_tiling_on_sc=False))
def _sc_aggr(*refs):
    embf = refs[0:4]
    edges = refs[4:12]
    ones_h, zrows_h, zrpt_h = refs[12:15]
    out1 = refs[15:23]
    fin = refs[23:31]
    acc, cnt = refs[31:33]
    (idxb0, idxb1, sadj0, sadj1, rows0, rows1, ones_v,
     zb, rcb, z1b, bidx, sem0, sem1) = refs[33:]

    c = lax.axis_index("c")
    s = lax.axis_index("s")
    row0 = s * RPT
    ebase = s * EPT

    pltpu.sync_copy(zrows_h, zb)
    pltpu.sync_copy(zrpt_h, z1b)
    pltpu.sync_copy(ones_h, ones_v)

    def zero_blk(b, _):
        r0 = row0 + b * RB
        pltpu.sync_copy(zb, acc.at[pl.ds(r0, RB), :])
        pltpu.sync_copy(z1b, cnt.at[pl.ds(r0, RB)])
        return 0
    lax.fori_loop(0, NBLK, zero_blk, 0)
    plsc.subcore_barrier()

    for layer in (1, 2):
        for ei, e in enumerate(ETYPES):
            rev_i = ETYPES.index(REV[e])
            edg = edges[ei]
            if layer == 1:
                tab = embf[NTYPES.index(e[0])]
                qsrc = jnp.broadcast_to(COL_OFF[e] // 32 + c, (16,))
            else:
                tab = out1[rev_i]

            def load_idx(i, idxb, sadj):
                off = ebase + i * C
                pltpu.sync_copy(edg.at[:, pl.ds(off, C)], idxb)
                if layer == 1:
                    def adj(j, _):
                        v = idxb[0, pl.ds(j * 16, 16)]
                        sadj[pl.ds(j * 16, 16)] = v * 4 + qsrc
                        return 0
                    lax.fori_loop(0, C // 16, adj, 0)

            if layer == 1:
                def start_g(idxb, sadj, rows, sem, tab=tab):
                    return pltpu.async_copy(tab.at[sadj], rows, sem)

                def wait_g(idxb, sadj, rows, sem, tab=tab):
                    pltpu.make_async_copy(tab.at[sadj], rows, sem).wait()
            else:
                def start_g(idxb, sadj, rows, sem, tab=tab):
                    return pltpu.async_copy(tab.at[c].at[idxb.at[0]],
                                            rows, sem)

                def wait_g(idxb, sadj, rows, sem, tab=tab):
                    pltpu.make_async_copy(tab.at[c].at[idxb.at[0]],
                                          rows, sem).wait()

            def drain(idxb, sadj, rows, sem):
                wait_g(idxb, sadj, rows, sem)
                pltpu.sync_copy(rows, acc.at[idxb.at[1]], add=True)
                pltpu.sync_copy(ones_v, cnt.at[idxb.at[1]], add=True)

            load_idx(0, idxb0, sadj0)
            start_g(idxb0, sadj0, rows0, sem0)

            def gs_pair(k, _):
                load_idx(2 * k + 1, idxb1, sadj1)
                start_g(idxb1, sadj1, rows1, sem1)
                drain(idxb0, sadj0, rows0, sem0)

                @pl.when(2 * k + 2 < NCHUNK)
                def _():
                    load_idx(2 * k + 2, idxb0, sadj0)
                    start_g(idxb0, sadj0, rows0, sem0)
                drain(idxb1, sadj1, rows1, sem1)
                return 0
            lax.fori_loop(0, NCHUNK // 2, gs_pair, 0)
            plsc.subcore_barrier()

            accb, bb = rows0, rows1
            if layer == 1:
                basef = embf[NTYPES.index(e[1])]
                qb = jnp.broadcast_to(COL_OFF[REV[e]] // 32 + c, (16,))

            def scale_blk(b, _, ei=ei, layer=layer):
                r0 = row0 + b * RB
                pltpu.sync_copy(cnt.at[pl.ds(r0, RB)], rcb.at[pl.ds(0, RB)])
                pltpu.sync_copy(acc.at[pl.ds(r0, RB), :],
                                accb.at[pl.ds(0, RB), :])
                if layer == 1:
                    def bix(j, _):
                        off = jnp.minimum(j * 16, RB - 16)
                        sl = lax.broadcasted_iota(jnp.int32, (16,), 0)
                        bidx[pl.ds(off, 16)] = (r0 + off) * 4 + sl * 4 + qb
                        return 0
                    lax.fori_loop(0, (RB + 15) // 16, bix, 0)
                    pltpu.async_copy(basef.at[bidx],
                                     bb.at[pl.ds(0, RB), :], sem0).wait()
                else:
                    pltpu.sync_copy(fin[ei].at[c].at[pl.ds(r0, RB), :],
                                    bb.at[pl.ds(0, RB), :])

                def rcp_body(j, _):
                    v = rcb[pl.ds(j * 16, 16)]
                    rcb[pl.ds(j * 16, 16)] = 1.0 / jnp.maximum(v, 1.0)
                    return 0
                lax.fori_loop(0, (RB + 15) // 16, rcp_body, 0)

                def row_body(r, _, layer=layer):
                    rcv = rcb[pl.ds(r, 16)][0]
                    if layer == 1:
                        bc = jnp.broadcast_to(rcv, (16,))
                        for h in (0, 16):
                            m = accb[r, pl.ds(h, 16)] * bc
                            accb[r, pl.ds(h, 16)] = m
                            bb[r, pl.ds(h, 16)] = bb[r, pl.ds(h, 16)] + m * 0.5
                    else:
                        bc3 = jnp.broadcast_to(rcv * (1.0 / 3.0), (16,))
                        for h in (0, 16):
                            a = accb[r, pl.ds(h, 16)]
                            bb[r, pl.ds(h, 16)] = bb[r, pl.ds(h, 16)] + a * bc3
                    return 0
                lax.fori_loop(0, RB, row_body, 0)

                if layer == 1:
                    pltpu.sync_copy(accb.at[pl.ds(0, RB), :],
                                    out1[ei].at[c].at[pl.ds(r0, RB), :])
                pltpu.sync_copy(bb.at[pl.ds(0, RB), :],
                                fin[ei].at[c].at[pl.ds(r0, RB), :])
                pltpu.sync_copy(zb, acc.at[pl.ds(r0, RB), :])
                pltpu.sync_copy(z1b, cnt.at[pl.ds(r0, RB)])
                return 0
            lax.fori_loop(0, NBLK, scale_blk, 0)
            plsc.subcore_barrier()


def kernel(user_emb, video_emb, publisher_emb, tag_emb,
           edge_index_uv, edge_index_up, edge_index_vu, edge_index_vt,
           edge_index_pu, edge_index_pt, edge_index_tv, edge_index_tp):
    embs = {'u': user_emb, 'v': video_emb, 'p': publisher_emb, 't': tag_emb}
    ei = {'uv': edge_index_uv, 'up': edge_index_up, 'vu': edge_index_vu,
          'vt': edge_index_vt, 'pu': edge_index_pu, 'pt': edge_index_pt,
          'tv': edge_index_tv, 'tp': edge_index_tp}

    pad_src = (jnp.arange(PADE, dtype=jnp.int32) % N)
    pad_dst = N + (jnp.arange(PADE, dtype=jnp.int32) % (NP - N))
    pad2 = jnp.stack([pad_src, pad_dst])

    embf = [jnp.pad(embs[t], ((0, NP - N), (0, 0))).reshape(NP * 4, HC)
            for t in NTYPES]
    edges = [jnp.concatenate([ei[e], pad2], axis=1) for e in ETYPES]

    ones_c = jnp.ones((C,), jnp.float32)
    zrows = jnp.zeros((RB, HC), jnp.float32)
    zrpt = jnp.zeros((RB,), jnp.float32)

    outs = _sc_aggr(*embf, *edges, ones_c, zrows, zrpt)
    fin = {e: outs[8 + i] for i, e in enumerate(ETYPES)}

    def cat(e1, e2):
        return jnp.concatenate([fin[e1][0, :N], fin[e1][1, :N],
                                fin[e2][0, :N], fin[e2][1, :N]], axis=1)

    return (cat('vu', 'pu'), cat('uv', 'tv'), cat('up', 'tp'),
            cat('vt', 'pt'))


# speedup vs baseline: 6.0363x; 1.0067x over previous
import functools

import jax
import jax.numpy as jnp
from jax import lax
from jax.experimental import pallas as pl
from jax.experimental.pallas import tpu as pltpu
from jax.experimental.pallas import tpu_sc as plsc

N = 50000
EMB = 128
H = EMB // 2
HC = H // 2
E = 500000

NTILE = 16
NP = 50048
RPT = NP // NTILE
RB = 136
NBLK = RPT // RB
RTAIL = N - (NP - RB)
G = 4
C = 160
NCHUNK = 196
EPT = NCHUNK * C
EP = EPT * NTILE
PADE = EP - E

ETYPES = ('uv', 'up', 'vu', 'vt', 'pu', 'pt', 'tv', 'tp')
REV = {'uv': 'vu', 'up': 'pu', 'vu': 'uv', 'vt': 'tv',
       'pu': 'up', 'pt': 'tp', 'tv': 'vt', 'tp': 'pt'}
COL_OFF = {'uv': 0, 'up': H, 'vu': 0, 'vt': H,
           'pu': 0, 'pt': H, 'tv': 0, 'tp': H}
NTYPES = ('u', 'v', 'p', 't')
HS = {'vu': 0, 'pu': 1, 'uv': 0, 'tv': 1, 'up': 0, 'tp': 1,
      'vt': 0, 'pt': 1}

_mesh = plsc.VectorSubcoreMesh(core_axis_name="c", subcore_axis_name="s")

_out_types = ([jax.ShapeDtypeStruct((2, NP, HC), jnp.float32)] * 8 +
              [jax.ShapeDtypeStruct((N, EMB), jnp.float32)] * 4)

_scratch = (
    [pltpu.VMEM_SHARED((NP, HC), jnp.float32),
     pltpu.VMEM_SHARED((NP,), jnp.float32)] +
    [pltpu.VMEM((2, C), jnp.int32)] * G +
    [pltpu.VMEM((C,), jnp.int32)] * G +
    [pltpu.VMEM((C, HC), jnp.float32)] * G +
    [pltpu.VMEM((C,), jnp.float32),
     pltpu.VMEM((RB, HC), jnp.float32),
     pltpu.VMEM((RB + 16,), jnp.float32),
     pltpu.VMEM((RB,), jnp.float32),
     pltpu.VMEM((RB,), jnp.int32)] +
    [pltpu.SemaphoreType.DMA] * G +
    [pltpu.SemaphoreType.DMA] * G
)


@functools.partial(pl.kernel, mesh=_mesh, out_type=_out_types,
                   scratch_types=_scratch,
                   compiler_params=pltpu.CompilerParams(
                       use_tc_tiling_on_sc=False))
def _sc_aggr(*refs):
    embf = refs[0:4]
    edges = refs[4:12]
    ones_h, zrows_h, zrpt_h = refs[12:15]
    out1 = refs[15:23]
    fin = refs[23:27]
    acc, cnt = refs[27:29]
    idxb = refs[29:29 + G]
    sadj = refs[33:33 + G]
    rows = refs[37:37 + G]
    (ones_v, zb, rcb, z1b, bidx) = refs[41:46]
    gsem = refs[46:46 + G]
    ssem = refs[50:50 + G]
    accb, bb = rows[0], rows[1]

    c = lax.axis_index("c")
    s = lax.axis_index("s")
    row0 = s * RPT
    ebase = s * EPT

    pltpu.sync_copy(zrows_h, zb)
    pltpu.sync_copy(zrpt_h, z1b)
    pltpu.sync_copy(ones_h, ones_v)

    def zero_blk(b, _):
        r0 = row0 + b * RB
        pltpu.sync_copy(zb, acc.at[pl.ds(r0, RB), :])
        pltpu.sync_copy(z1b, cnt.at[pl.ds(r0, RB)])
        return 0
    lax.fori_loop(0, NBLK, zero_blk, 0)
    plsc.subcore_barrier()

    for layer in (1, 2):
        for ei, e in enumerate(ETYPES):
            rev_i = ETYPES.index(REV[e])
            edg = edges[ei]
            tgt = NTYPES.index(e[1])
            col0 = HS[e] * H + c * HC
            if layer == 1:
                tab = embf[NTYPES.index(e[0])]
                qsrc = jnp.broadcast_to(COL_OFF[e] // 32 + c, (16,))
            else:
                tab = out1[rev_i]

            def fire(i, g, tab=tab, layer=layer, qsrc=(qsrc if layer == 1
                                                       else None)):
                off = ebase + i * C
                pltpu.sync_copy(edg.at[:, pl.ds(off, C)], idxb[g])
                if layer == 1:
                    def adj(j, _):
                        v = idxb[g][0, pl.ds(j * 16, 16)]
                        sadj[g][pl.ds(j * 16, 16)] = v * 4 + qsrc
                        return 0
                    lax.fori_loop(0, C // 16, adj, 0)
                    pltpu.async_copy(tab.at[sadj[g]], rows[g], gsem[g])
                else:
                    pltpu.async_copy(tab.at[c].at[idxb[g].at[0]],
                                     rows[g], gsem[g])

            def wait_g(g, tab=tab, layer=layer):
                if layer == 1:
                    pltpu.make_async_copy(tab.at[sadj[g]], rows[g],
                                          gsem[g]).wait()
                else:
                    pltpu.make_async_copy(tab.at[c].at[idxb[g].at[0]],
                                          rows[g], gsem[g]).wait()

            def fire_scatter(g):
                pltpu.async_copy(rows[g], acc.at[idxb[g].at[1]], ssem[g],
                                 add=True)
                pltpu.async_copy(ones_v, cnt.at[idxb[g].at[1]], ssem[g],
                                 add=True)

            def drain_scatter(g):
                pltpu.make_async_copy(rows[g], acc.at[idxb[g].at[1]],
                                      ssem[g]).wait()
                pltpu.make_async_copy(ones_v, cnt.at[idxb[g].at[1]],
                                      ssem[g]).wait()

            for g in range(G):
                fire(g, g)

            def gs_group(k, _):
                base_i = G * k
                for g in range(G):
                    wait_g(g)
                    fire_scatter(g)

                @pl.when(base_i + G < NCHUNK)
                def _():
                    for g in range(G):
                        drain_scatter(g)
                        fire(base_i + G + g, g)
                return 0
            lax.fori_loop(0, NCHUNK // G, gs_group, 0)
            for g in range(G):
                drain_scatter(g)
            plsc.subcore_barrier()

            if layer == 1:
                basef = embf[tgt]
                qb = jnp.broadcast_to(COL_OFF[REV[e]] // 32 + c, (16,))

            def scale_blk(b, _, ei=ei, tgt=tgt, col0=col0, layer=layer):
                r0 = row0 + b * RB
                pltpu.sync_copy(cnt.at[pl.ds(r0, RB)], rcb.at[pl.ds(0, RB)])
                pltpu.sync_copy(acc.at[pl.ds(r0, RB), :],
                                accb.at[pl.ds(0, RB), :])
                if layer == 1:
                    def bix(j, _):
                        off = jnp.minimum(j * 16, RB - 16)
                        sl = lax.broadcasted_iota(jnp.int32, (16,), 0)
                        bidx[pl.ds(off, 16)] = (r0 + off) * 4 + sl * 4 + qb
                        return 0
                    lax.fori_loop(0, (RB + 15) // 16, bix, 0)
                    pltpu.async_copy(basef.at[bidx],
                                     bb.at[pl.ds(0, RB), :], gsem[0]).wait()
                else:
                    @pl.when(r0 + RB <= N)
                    def _():
                        pltpu.sync_copy(
                            fin[tgt].at[pl.ds(r0, RB), pl.ds(col0, HC)],
                            bb.at[pl.ds(0, RB), :])

                    @pl.when(r0 + RB > N)
                    def _():
                        pltpu.sync_copy(
                            fin[tgt].at[pl.ds(r0, RTAIL), pl.ds(col0, HC)],
                            bb.at[pl.ds(0, RTAIL), :])

                def rcp_body(j, _):
                    v = rcb[pl.ds(j * 16, 16)]
                    rcb[pl.ds(j * 16, 16)] = 1.0 / jnp.maximum(v, 1.0)
                    return 0
                lax.fori_loop(0, (RB + 15) // 16, rcp_body, 0)

                def row_body(r, _, layer=layer):
                    rcv = rcb[pl.ds(r, 16)][0]
                    if layer == 1:
                        bc = jnp.broadcast_to(rcv, (16,))
                        for h in (0, 16):
                            m = accb[r, pl.ds(h, 16)] * bc
                            accb[r, pl.ds(h, 16)] = m
                            bb[r, pl.ds(h, 16)] = bb[r, pl.ds(h, 16)] + m * 0.5
                    else:
                        bc3 = jnp.broadcast_to(rcv * (1.0 / 3.0), (16,))
                        for h in (0, 16):
                            a = accb[r, pl.ds(h, 16)]
                            bb[r, pl.ds(h, 16)] = bb[r, pl.ds(h, 16)] + a * bc3
                    return 0
                lax.fori_loop(0, RB, row_body, 0)

                if layer == 1:
                    pltpu.sync_copy(accb.at[pl.ds(0, RB), :],
                                    out1[ei].at[c].at[pl.ds(r0, RB), :])

                @pl.when(r0 + RB <= N)
                def _():
                    pltpu.sync_copy(
                        bb.at[pl.ds(0, RB), :],
                        fin[tgt].at[pl.ds(r0, RB), pl.ds(col0, HC)])

                @pl.when(r0 + RB > N)
                def _():
                    pltpu.sync_copy(
                        bb.at[pl.ds(0, RTAIL), :],
                        fin[tgt].at[pl.ds(r0, RTAIL), pl.ds(col0, HC)])

                pltpu.sync_copy(zb, acc.at[pl.ds(r0, RB), :])
                pltpu.sync_copy(z1b, cnt.at[pl.ds(r0, RB)])
                return 0
            lax.fori_loop(0, NBLK, scale_blk, 0)
            plsc.subcore_barrier()


def kernel(user_emb, video_emb, publisher_emb, tag_emb,
           edge_index_uv, edge_index_up, edge_index_vu, edge_index_vt,
           edge_index_pu, edge_index_pt, edge_index_tv, edge_index_tp):
    embs = {'u': user_emb, 'v': video_emb, 'p': publisher_emb, 't': tag_emb}
    ei = {'uv': edge_index_uv, 'up': edge_index_up, 'vu': edge_index_vu,
          'vt': edge_index_vt, 'pu': edge_index_pu, 'pt': edge_index_pt,
          'tv': edge_index_tv, 'tp': edge_index_tp}

    embf = [jnp.pad(embs[t], ((0, NP - N), (0, 0))).reshape(NP * 4, HC)
            for t in NTYPES]
    edges = [jnp.pad(ei[e], ((0, 0), (0, PADE)), constant_values=N)
             for e in ETYPES]

    ones_c = jnp.ones((C,), jnp.float32)
    zrows = jnp.zeros((RB, HC), jnp.float32)
    zrpt = jnp.zeros((RB,), jnp.float32)

    outs = _sc_aggr(*embf, *edges, ones_c, zrows, zrpt)
    return (outs[8], outs[9], outs[10], outs[11])


# speedup vs baseline: 6.6240x; 1.0974x over previous
import functools

import jax
import jax.numpy as jnp
from jax import lax
from jax.experimental import pallas as pl
from jax.experimental.pallas import tpu as pltpu
from jax.experimental.pallas import tpu_sc as plsc

N = 50000
EMB = 128
H = EMB // 2
HC = H // 2
E = 500000

NTILE = 16
NP = 50048
RPT = NP // NTILE
RB = 136
NBLK = RPT // RB
RTAIL = N - (NP - RB)
G = 2
C = 320
NCHUNK = 98
EPT = NCHUNK * C
EP = EPT * NTILE
PADE = EP - E

ETYPES = ('uv', 'up', 'vu', 'vt', 'pu', 'pt', 'tv', 'tp')
REV = {'uv': 'vu', 'up': 'pu', 'vu': 'uv', 'vt': 'tv',
       'pu': 'up', 'pt': 'tp', 'tv': 'vt', 'tp': 'pt'}
COL_OFF = {'uv': 0, 'up': H, 'vu': 0, 'vt': H,
           'pu': 0, 'pt': H, 'tv': 0, 'tp': H}
NTYPES = ('u', 'v', 'p', 't')
HS = {'vu': 0, 'pu': 1, 'uv': 0, 'tv': 1, 'up': 0, 'tp': 1,
      'vt': 0, 'pt': 1}

_mesh = plsc.VectorSubcoreMesh(core_axis_name="c", subcore_axis_name="s")

_out_types = ([jax.ShapeDtypeStruct((2, NP, HC), jnp.float32)] * 8 +
              [jax.ShapeDtypeStruct((N, EMB), jnp.float32)] * 4 +
              [jax.ShapeDtypeStruct((8, NP), jnp.float32)])

_scratch = (
    [pltpu.VMEM_SHARED((NP, HC), jnp.float32),
     pltpu.VMEM_SHARED((NP,), jnp.float32)] +
    [pltpu.VMEM((2, C), jnp.int32)] * G +
    [pltpu.VMEM((C,), jnp.int32)] * G +
    [pltpu.VMEM((C, HC), jnp.float32)] * G +
    [pltpu.VMEM((C,), jnp.float32),
     pltpu.VMEM((RB, HC), jnp.float32),
     pltpu.VMEM((RB + 16,), jnp.float32),
     pltpu.VMEM((RB,), jnp.float32),
     pltpu.VMEM((RB,), jnp.int32)] +
    [pltpu.SemaphoreType.DMA] * G +
    [pltpu.SemaphoreType.DMA] * G
)


@functools.partial(pl.kernel, mesh=_mesh, out_type=_out_types,
                   scratch_types=_scratch,
                   compiler_params=pltpu.CompilerParams(
                       use_tc_tiling_on_sc=False))
def _sc_aggr(*refs):
    embf = refs[0:4]
    edges = refs[4:12]
    ones_h, zrows_h, zrpt_h = refs[12:15]
    out1 = refs[15:23]
    fin = refs[23:27]
    recip_h = refs[27]
    acc, cnt = refs[28:30]
    idxb = refs[30:30 + G]
    sadj = refs[30 + G:30 + 2 * G]
    rows = refs[30 + 2 * G:30 + 3 * G]
    (ones_v, zb, rcb, z1b, bidx) = refs[30 + 3 * G:35 + 3 * G]
    gsem = refs[35 + 3 * G:35 + 4 * G]
    ssem = refs[35 + 4 * G:35 + 5 * G]
    accb, bb = rows[0], rows[1]

    c = lax.axis_index("c")
    s = lax.axis_index("s")
    row0 = s * RPT
    ebase = s * EPT

    pltpu.sync_copy(zrows_h, zb)
    pltpu.sync_copy(zrpt_h, z1b)
    pltpu.sync_copy(ones_h, ones_v)

    def zero_blk(b, _):
        r0 = row0 + b * RB
        pltpu.sync_copy(zb, acc.at[pl.ds(r0, RB), :])
        pltpu.sync_copy(z1b, cnt.at[pl.ds(r0, RB)])
        return 0
    lax.fori_loop(0, NBLK, zero_blk, 0)
    plsc.subcore_barrier()

    for layer in (1, 2):
        for ei, e in enumerate(ETYPES):
            rev_i = ETYPES.index(REV[e])
            edg = edges[ei]
            tgt = NTYPES.index(e[1])
            col0 = HS[e] * H + c * HC
            if layer == 1:
                tab = embf[NTYPES.index(e[0])]
                qsrc = jnp.broadcast_to(COL_OFF[e] // 32 + c, (16,))
            else:
                tab = out1[rev_i]

            def fire(i, g, tab=tab, layer=layer, qsrc=(qsrc if layer == 1
                                                       else None)):
                off = ebase + i * C
                pltpu.sync_copy(edg.at[:, pl.ds(off, C)], idxb[g])
                if layer == 1:
                    def adj(j, _):
                        v = idxb[g][0, pl.ds(j * 16, 16)]
                        sadj[g][pl.ds(j * 16, 16)] = v * 4 + qsrc
                        return 0
                    lax.fori_loop(0, C // 16, adj, 0)
                    pltpu.async_copy(tab.at[sadj[g]], rows[g], gsem[g])
                else:
                    pltpu.async_copy(tab.at[c].at[idxb[g].at[0]],
                                     rows[g], gsem[g])

            def wait_g(g, tab=tab, layer=layer):
                if layer == 1:
                    pltpu.make_async_copy(tab.at[sadj[g]], rows[g],
                                          gsem[g]).wait()
                else:
                    pltpu.make_async_copy(tab.at[c].at[idxb[g].at[0]],
                                          rows[g], gsem[g]).wait()

            def fire_scatter(g, layer=layer):
                pltpu.async_copy(rows[g], acc.at[idxb[g].at[1]], ssem[g],
                                 add=True)
                if layer == 1:
                    pltpu.async_copy(ones_v, cnt.at[idxb[g].at[1]], ssem[g],
                                     add=True)

            def drain_scatter(g, layer=layer):
                pltpu.make_async_copy(rows[g], acc.at[idxb[g].at[1]],
                                      ssem[g]).wait()
                if layer == 1:
                    pltpu.make_async_copy(ones_v, cnt.at[idxb[g].at[1]],
                                          ssem[g]).wait()

            fire(0, 0)

            def gs_pair(k, _):
                @pl.when(k > 0)
                def _():
                    drain_scatter(1)
                fire(2 * k + 1, 1)
                wait_g(0)
                fire_scatter(0)
                wait_g(1)
                fire_scatter(1)

                @pl.when(2 * k + 2 < NCHUNK)
                def _():
                    drain_scatter(0)
                    fire(2 * k + 2, 0)
                return 0
            lax.fori_loop(0, NCHUNK // G, gs_pair, 0)
            drain_scatter(0)
            drain_scatter(1)
            plsc.subcore_barrier()

            if layer == 1:
                basef = embf[tgt]
                qb = jnp.broadcast_to(COL_OFF[REV[e]] // 32 + c, (16,))

            def scale_blk(b, _, ei=ei, tgt=tgt, col0=col0, layer=layer):
                r0 = row0 + b * RB
                if layer == 1:
                    pltpu.sync_copy(cnt.at[pl.ds(r0, RB)],
                                    rcb.at[pl.ds(0, RB)])
                else:
                    pltpu.sync_copy(recip_h.at[ei].at[pl.ds(r0, RB)],
                                    rcb.at[pl.ds(0, RB)])
                pltpu.sync_copy(acc.at[pl.ds(r0, RB), :],
                                accb.at[pl.ds(0, RB), :])
                if layer == 1:
                    def bix(j, _):
                        off = jnp.minimum(j * 16, RB - 16)
                        sl = lax.broadcasted_iota(jnp.int32, (16,), 0)
                        bidx[pl.ds(off, 16)] = (r0 + off) * 4 + sl * 4 + qb
                        return 0
                    lax.fori_loop(0, (RB + 15) // 16, bix, 0)
                    pltpu.async_copy(basef.at[bidx],
                                     bb.at[pl.ds(0, RB), :], gsem[0]).wait()
                else:
                    @pl.when(r0 + RB <= N)
                    def _():
                        pltpu.sync_copy(
                            fin[tgt].at[pl.ds(r0, RB), pl.ds(col0, HC)],
                            bb.at[pl.ds(0, RB), :])

                    @pl.when(r0 + RB > N)
                    def _():
                        pltpu.sync_copy(
                            fin[tgt].at[pl.ds(r0, RTAIL), pl.ds(col0, HC)],
                            bb.at[pl.ds(0, RTAIL), :])

                if layer == 1:
                    def rcp_body(j, _):
                        v = rcb[pl.ds(j * 16, 16)]
                        rcb[pl.ds(j * 16, 16)] = 1.0 / jnp.maximum(v, 1.0)
                        return 0
                    lax.fori_loop(0, (RB + 15) // 16, rcp_body, 0)
                    pltpu.sync_copy(rcb.at[pl.ds(0, RB)],
                                    recip_h.at[ei].at[pl.ds(r0, RB)])

                def row_body(r, _, layer=layer):
                    rcv = rcb[pl.ds(r, 16)][0]
                    if layer == 1:
                        bc = jnp.broadcast_to(rcv, (16,))
                        for h in (0, 16):
                            m = accb[r, pl.ds(h, 16)] * bc
                            accb[r, pl.ds(h, 16)] = m
                            bb[r, pl.ds(h, 16)] = bb[r, pl.ds(h, 16)] + m * 0.5
                    else:
                        bc3 = jnp.broadcast_to(rcv * (1.0 / 3.0), (16,))
                        for h in (0, 16):
                            a = accb[r, pl.ds(h, 16)]
                            bb[r, pl.ds(h, 16)] = bb[r, pl.ds(h, 16)] + a * bc3
                    return 0
                lax.fori_loop(0, RB, row_body, 0)

                if layer == 1:
                    pltpu.sync_copy(accb.at[pl.ds(0, RB), :],
                                    out1[ei].at[c].at[pl.ds(r0, RB), :])

                @pl.when(r0 + RB <= N)
                def _():
                    pltpu.sync_copy(
                        bb.at[pl.ds(0, RB), :],
                        fin[tgt].at[pl.ds(r0, RB), pl.ds(col0, HC)])

                @pl.when(r0 + RB > N)
                def _():
                    pltpu.sync_copy(
                        bb.at[pl.ds(0, RTAIL), :],
                        fin[tgt].at[pl.ds(r0, RTAIL), pl.ds(col0, HC)])

                pltpu.sync_copy(zb, acc.at[pl.ds(r0, RB), :])
                if layer == 1:
                    pltpu.sync_copy(z1b, cnt.at[pl.ds(r0, RB)])
                return 0
            lax.fori_loop(0, NBLK, scale_blk, 0)
            plsc.subcore_barrier()


def kernel(user_emb, video_emb, publisher_emb, tag_emb,
           edge_index_uv, edge_index_up, edge_index_vu, edge_index_vt,
           edge_index_pu, edge_index_pt, edge_index_tv, edge_index_tp):
    embs = {'u': user_emb, 'v': video_emb, 'p': publisher_emb, 't': tag_emb}
    ei = {'uv': edge_index_uv, 'up': edge_index_up, 'vu': edge_index_vu,
          'vt': edge_index_vt, 'pu': edge_index_pu, 'pt': edge_index_pt,
          'tv': edge_index_tv, 'tp': edge_index_tp}

    embf = [jnp.pad(embs[t], ((0, NP - N), (0, 0))).reshape(NP * 4, HC)
            for t in NTYPES]
    edges = [jnp.pad(ei[e], ((0, 0), (0, PADE)), constant_values=N)
             for e in ETYPES]

    ones_c = jnp.ones((C,), jnp.float32)
    zrows = jnp.zeros((RB, HC), jnp.float32)
    zrpt = jnp.zeros((RB,), jnp.float32)

    outs = _sc_aggr(*embf, *edges, ones_c, zrows, zrpt)
    return (outs[8], outs[9], outs[10], outs[11])
